# baseline (speedup 1.0000x reference)
import functools

import numpy as np
import jax
import jax.numpy as jnp
from jax import lax
from jax.experimental import pallas as pl
from jax.experimental.pallas import tpu as pltpu
from jax.experimental.pallas import tpu_sc as plsc

DIM = 16
NPROP = 26
NOBJ = 21
NVAL = 100000
BATCH = 1024
ROWS = BATCH * NOBJ * NPROP
FLAT = NOBJ * NPROP * DIM

NC, NS, L = 2, 16, 16
NW = NC * NS
RPW = ROWS // NW
STEP = 96
SPW = RPW // STEP
KSTEP = 13
NSUP = SPW // KSTEP
SUP = KSTEP * STEP


_VOCAB = 1 + NVAL * NPROP
_RL_BC = 2048
_RL_GRID = -(-_VOCAB // _RL_BC)
_VPAD = _RL_GRID * _RL_BC


def _tc_relayout(tableT):
    def body(in_ref, out_ref):
        a3 = in_ref[...].reshape(DIM, _RL_BC // 8, 8)
        for s in range(8):
            out_ref[:, s * DIM:(s + 1) * DIM] = a3[:, :, s].T

    return pl.pallas_call(
        body,
        grid=(_RL_GRID,),
        in_specs=[pl.BlockSpec((DIM, _RL_BC), lambda i: (0, i))],
        out_specs=pl.BlockSpec((_RL_BC // 8, 128), lambda i: (i, 0)),
        out_shape=jax.ShapeDtypeStruct((_VPAD // 8 * DIM // 16, 128),
                                       jnp.float32),
    )(tableT)


def _sc_gather(x3d, table):
    mesh = plsc.VectorSubcoreMesh(
        core_axis_name="c", subcore_axis_name="s",
        num_cores=NC, num_subcores=NS)

    @functools.partial(
        pl.kernel,
        out_type=jax.ShapeDtypeStruct((ROWS, DIM), jnp.float32),
        name="sc_embed_gather",
        mesh=mesh,
        scratch_types=[
            pltpu.VMEM((SPW, STEP), jnp.int32),
            pltpu.VMEM((SUP, DIM), jnp.float32),
            pltpu.SemaphoreType.DMA,
        ],
        compiler_params=pltpu.CompilerParams(use_tc_tiling_on_sc=False),
    )
    def k(x_hbm, table_hbm, out_hbm, idx_v, rows_v, sem):
        wid = lax.axis_index("s") * NC + lax.axis_index("c")
        row_base = wid * RPW
        pltpu.sync_copy(x_hbm.at[wid], idx_v)

        lanes = lax.iota(jnp.int32, L)

        def to_idx(i, carry):
            for j in range(STEP // L):
                r0 = row_base + i * STEP + j * L
                prop = (r0 + lanes) % NPROP
                v = idx_v[i, pl.ds(j * L, L)]
                idx_v[i, pl.ds(j * L, L)] = v + prop * NVAL
            return carry

        lax.fori_loop(0, SPW, to_idx, 0)

        def sup(s, carry):
            cps = [
                pltpu.async_copy(
                    table_hbm.at[idx_v.at[s * KSTEP + j]],
                    rows_v.at[pl.ds(j * STEP, STEP)],
                    sem)
                for j in range(KSTEP)
            ]
            for c in cps:
                c.wait()
            pltpu.sync_copy(rows_v, out_hbm.at[pl.ds(row_base + s * SUP, SUP)])
            return carry

        lax.fori_loop(0, NSUP, sup, 0)

    return k(x3d, table)


_EG = (np.arange(NOBJ * NPROP)[:, None] // NPROP
       == np.arange(NOBJ)[None, :]).astype(np.float32)
_E16 = (np.arange(NOBJ)[:, None]
        == np.arange(FLAT)[None, :] // (NPROP * DIM)).astype(np.float32)
_E546 = _EG.T.copy()

_B_BLK = 128


def _tc_finish(raw2, x2, pattern, absent_t):
    grid = (BATCH // _B_BLK,)

    def body(raw_ref, x_ref, pat_ref, abs_ref, eg_ref, e16_ref, e546_ref,
             out_ref, pad_ref):
        xf = x_ref[...].astype(jnp.float32)
        sums = jnp.dot(xf, eg_ref[...], preferred_element_type=jnp.float32)
        padf = (sums == 0.0).astype(jnp.float32)
        m16 = jnp.dot(padf, e16_ref[...], preferred_element_type=jnp.float32)
        m546 = jnp.dot(padf, e546_ref[...], preferred_element_type=jnp.float32)
        emb = raw_ref[...] + pat_ref[...]
        out_ref[...] = emb * (1.0 - m16) + abs_ref[...] * m16
        pad_ref[...] = m546 > 0.5

    out2, padflat = pl.pallas_call(
        body,
        grid=grid,
        in_specs=[
            pl.BlockSpec((_B_BLK, FLAT), lambda i: (i, 0)),
            pl.BlockSpec((_B_BLK, NOBJ * NPROP), lambda i: (i, 0)),
            pl.BlockSpec((1, FLAT), lambda i: (0, 0)),
            pl.BlockSpec((1, FLAT), lambda i: (0, 0)),
            pl.BlockSpec((NOBJ * NPROP, NOBJ), lambda i: (0, 0)),
            pl.BlockSpec((NOBJ, FLAT), lambda i: (0, 0)),
            pl.BlockSpec((NOBJ, NOBJ * NPROP), lambda i: (0, 0)),
        ],
        out_specs=[
            pl.BlockSpec((_B_BLK, FLAT), lambda i: (i, 0)),
            pl.BlockSpec((_B_BLK, NOBJ * NPROP), lambda i: (i, 0)),
        ],
        out_shape=[
            jax.ShapeDtypeStruct((BATCH, FLAT), jnp.float32),
            jax.ShapeDtypeStruct((BATCH, NOBJ * NPROP), jnp.bool_),
        ],
    )(raw2, x2, pattern, absent_t, jnp.asarray(_EG), jnp.asarray(_E16),
      jnp.asarray(_E546))
    return out2, padflat


def kernel(table, mark_features, mark_objects, mark_absent, x):
    x3d = x.reshape(NW, SPW, STEP)
    table_wide = _tc_relayout(table.T)
    table_rm = table_wide.reshape(_VPAD, DIM)
    raw = _sc_gather(x3d, table_rm)
    raw2 = raw.reshape(BATCH, FLAT)

    pattern = (mark_objects.reshape(NOBJ, 1, DIM)
               + mark_features.reshape(1, NPROP, DIM)).reshape(1, FLAT)
    absent_t = jnp.tile(mark_absent.reshape(1, DIM), (1, NOBJ * NPROP))
    x2 = x.reshape(BATCH, NOBJ * NPROP)

    out2, padflat = _tc_finish(raw2, x2, pattern, absent_t)
    return out2.reshape(BATCH, NOBJ * NPROP, DIM), padflat


# speedup vs baseline: 4.4057x; 4.4057x over previous
import functools

import numpy as np
import jax
import jax.numpy as jnp
from jax import lax
from jax.experimental import pallas as pl
from jax.experimental.pallas import tpu as pltpu
from jax.experimental.pallas import tpu_sc as plsc

DIM = 16
NPROP = 26
NOBJ = 21
NVAL = 100000
BATCH = 1024
ROWS = BATCH * NOBJ * NPROP
FLAT = NOBJ * NPROP * DIM

NC, NS, L = 2, 16, 16
NW = NC * NS
RPW = ROWS // NW
STEP = 96
SPW = RPW // STEP
KSTEP = 13
NSUP = SPW // KSTEP
SUP = KSTEP * STEP


_VOCAB = 1 + NVAL * NPROP
_NCOLS = _VOCAB // 128
_TAILBASE = _NCOLS * 128
_TAILPAD = _TAILBASE + 128
_WROWS = 325120
_VGATHER = _WROWS * 128 // DIM
_CPW_LO = _NCOLS // NW
_CPW_REM = _NCOLS - _CPW_LO * NW


def _sc_relayout(tableT, tail_wide):
    mesh = plsc.VectorSubcoreMesh(
        core_axis_name="c", subcore_axis_name="s",
        num_cores=NC, num_subcores=NS)

    @functools.partial(
        pl.kernel,
        out_type=jax.ShapeDtypeStruct((_WROWS, 128), jnp.float32),
        name="sc_table_relayout",
        mesh=mesh,
        scratch_types=[
            pltpu.VMEM((2, 2, 8, 128), jnp.float32),
            pltpu.VMEM((2, 2, 8, 128), jnp.float32),
            pltpu.SemaphoreType.DMA,
            pltpu.SemaphoreType.DMA,
        ],
        compiler_params=pltpu.CompilerParams(use_tc_tiling_on_sc=True,
                                             needs_layout_passes=False),
    )
    def k(t_hbm, tail_hbm, out_hbm, bin_v, bout_v, gsem, osem):
        wid = lax.axis_index("s") * NC + lax.axis_index("c")
        start = wid * _CPW_LO + jnp.minimum(wid, _CPW_REM)
        count = _CPW_LO + (wid < _CPW_REM).astype(jnp.int32)

        iot = lax.iota(jnp.int32, L)
        tv = iot // 8
        sv = iot % 8

        def fire_in(c, b):
            for h in range(2):
                pltpu.async_copy(
                    t_hbm.at[pl.ds(8 * h, 8), pl.ds(c * 128, 128)],
                    bin_v.at[b, h], gsem)

        def drain(sem, ref):
            pltpu.make_async_copy(t_hbm.at[pl.ds(0, 8), pl.ds(0, 128)],
                                  ref, sem).wait()

        def shuffle(b):
            for l in range(128):
                row = plsc.load_gather(
                    bin_v.at[b], [tv, sv, jnp.full((L,), l, jnp.int32)])
                bout_v[b, l // 64, (l // 8) % 8, pl.ds((l % 8) * 16, 16)] = row

        def fire_out(c, b):
            for h in range(2):
                pltpu.async_copy(
                    bout_v.at[b, h],
                    out_hbm.at[pl.ds(16 * c + 8 * h, 8), :], osem)

        def body(i, carry):
            c = start + i
            for h in range(2):
                pltpu.sync_copy(
                    t_hbm.at[pl.ds(8 * h, 8), pl.ds(c * 128, 128)],
                    bin_v.at[0, h])
            shuffle(0)
            for h in range(2):
                pltpu.sync_copy(bout_v.at[0, h],
                                out_hbm.at[pl.ds(16 * c + 8 * h, 8), :])
            return carry

        lax.fori_loop(0, count, body, 0)

        @pl.when(wid == NW - 1)
        def _():
            for h in range(2):
                pltpu.sync_copy(tail_hbm.at[pl.ds(8 * h, 8), :],
                                bin_v.at[0, h])
                pltpu.sync_copy(bin_v.at[0, h],
                                out_hbm.at[pl.ds(_TAILPAD // 8 + 8 * h, 8), :])

    return k(tableT, tail_wide)


def _sc_gather(x3d, table):
    mesh = plsc.VectorSubcoreMesh(
        core_axis_name="c", subcore_axis_name="s",
        num_cores=NC, num_subcores=NS)

    @functools.partial(
        pl.kernel,
        out_type=jax.ShapeDtypeStruct((ROWS, DIM), jnp.float32),
        name="sc_embed_gather",
        mesh=mesh,
        scratch_types=[
            pltpu.VMEM((SPW, STEP), jnp.int32),
            pltpu.VMEM((SUP, DIM), jnp.float32),
            pltpu.SemaphoreType.DMA,
        ],
        compiler_params=pltpu.CompilerParams(use_tc_tiling_on_sc=False),
    )
    def k(x_hbm, table_hbm, out_hbm, idx_v, rows_v, sem):
        wid = lax.axis_index("s") * NC + lax.axis_index("c")
        row_base = wid * RPW
        pltpu.sync_copy(x_hbm.at[wid], idx_v)

        lanes = lax.iota(jnp.int32, L)

        def to_idx(i, carry):
            for j in range(STEP // L):
                r0 = row_base + i * STEP + j * L
                prop = (r0 + lanes) % NPROP
                v = idx_v[i, pl.ds(j * L, L)] + prop * NVAL
                idx_v[i, pl.ds(j * L, L)] = jnp.where(
                    v >= _TAILBASE, v + (_TAILPAD - _TAILBASE), v)
            return carry

        lax.fori_loop(0, SPW, to_idx, 0)

        def sup(s, carry):
            cps = [
                pltpu.async_copy(
                    table_hbm.at[idx_v.at[s * KSTEP + j]],
                    rows_v.at[pl.ds(j * STEP, STEP)],
                    sem)
                for j in range(KSTEP)
            ]
            for c in cps:
                c.wait()
            pltpu.sync_copy(rows_v, out_hbm.at[pl.ds(row_base + s * SUP, SUP)])
            return carry

        lax.fori_loop(0, NSUP, sup, 0)

    return k(x3d, table)


_EG = (np.arange(NOBJ * NPROP)[:, None] // NPROP
       == np.arange(NOBJ)[None, :]).astype(np.float32)
_E16 = (np.arange(NOBJ)[:, None]
        == np.arange(FLAT)[None, :] // (NPROP * DIM)).astype(np.float32)
_E546 = _EG.T.copy()

_B_BLK = 128


def _tc_finish(raw2, x2, pattern, absent_t):
    grid = (BATCH // _B_BLK,)

    def body(raw_ref, x_ref, pat_ref, abs_ref, eg_ref, e16_ref, e546_ref,
             out_ref, pad_ref):
        xf = x_ref[...].astype(jnp.float32)
        sums = jnp.dot(xf, eg_ref[...], preferred_element_type=jnp.float32)
        padf = (sums == 0.0).astype(jnp.float32)
        m16 = jnp.dot(padf, e16_ref[...], preferred_element_type=jnp.float32)
        m546 = jnp.dot(padf, e546_ref[...], preferred_element_type=jnp.float32)
        emb = raw_ref[...] + pat_ref[...]
        out_ref[...] = emb * (1.0 - m16) + abs_ref[...] * m16
        pad_ref[...] = m546 > 0.5

    out2, padflat = pl.pallas_call(
        body,
        grid=grid,
        in_specs=[
            pl.BlockSpec((_B_BLK, FLAT), lambda i: (i, 0)),
            pl.BlockSpec((_B_BLK, NOBJ * NPROP), lambda i: (i, 0)),
            pl.BlockSpec((1, FLAT), lambda i: (0, 0)),
            pl.BlockSpec((1, FLAT), lambda i: (0, 0)),
            pl.BlockSpec((NOBJ * NPROP, NOBJ), lambda i: (0, 0)),
            pl.BlockSpec((NOBJ, FLAT), lambda i: (0, 0)),
            pl.BlockSpec((NOBJ, NOBJ * NPROP), lambda i: (0, 0)),
        ],
        out_specs=[
            pl.BlockSpec((_B_BLK, FLAT), lambda i: (i, 0)),
            pl.BlockSpec((_B_BLK, NOBJ * NPROP), lambda i: (i, 0)),
        ],
        out_shape=[
            jax.ShapeDtypeStruct((BATCH, FLAT), jnp.float32),
            jax.ShapeDtypeStruct((BATCH, NOBJ * NPROP), jnp.bool_),
        ],
    )(raw2, x2, pattern, absent_t, jnp.asarray(_EG), jnp.asarray(_E16),
      jnp.asarray(_E546))
    return out2, padflat


def kernel(table, mark_features, mark_objects, mark_absent, x):
    x3d = x.reshape(NW, SPW, STEP)
    tail_wide = jnp.pad(table[_TAILBASE:, :],
                        ((0, 128 - (_VOCAB - _TAILBASE)), (0, 0))
                        ).reshape(DIM, 128)
    table_wide = _sc_relayout(table.T, tail_wide)
    table_rm = table_wide.reshape(_VGATHER, DIM)
    raw = _sc_gather(x3d, table_rm)
    raw2 = raw.reshape(BATCH, FLAT)

    pattern = (mark_objects.reshape(NOBJ, 1, DIM)
               + mark_features.reshape(1, NPROP, DIM)).reshape(1, FLAT)
    absent_t = jnp.tile(mark_absent.reshape(1, DIM), (1, NOBJ * NPROP))
    x2 = x.reshape(BATCH, NOBJ * NPROP)

    out2, padflat = _tc_finish(raw2, x2, pattern, absent_t)
    return out2.reshape(BATCH, NOBJ * NPROP, DIM), padflat


# speedup vs baseline: 6.3756x; 1.4471x over previous
import functools

import numpy as np
import jax
import jax.numpy as jnp
from jax import lax
from jax.experimental import pallas as pl
from jax.experimental.pallas import tpu as pltpu
from jax.experimental.pallas import tpu_sc as plsc

DIM = 16
NPROP = 26
NOBJ = 21
NVAL = 100000
BATCH = 1024
ROWS = BATCH * NOBJ * NPROP
FLAT = NOBJ * NPROP * DIM

NC, NS, L = 2, 16, 16
NW = NC * NS
RPW = ROWS // NW
STEP = 96
SPW = RPW // STEP
KSTEP = 13
NSUP = SPW // KSTEP
SUP = KSTEP * STEP


_VOCAB = 1 + NVAL * NPROP
_NCOLS = _VOCAB // 128
_TAILBASE = _NCOLS * 128
_TAILPAD = _TAILBASE + 128
_WROWS = 325120
_VGATHER = _WROWS * 128 // DIM
_CPW_LO = _NCOLS // NW
_CPW_REM = _NCOLS - _CPW_LO * NW


def _sc_relayout(tableT, tail_wide):
    mesh = plsc.VectorSubcoreMesh(
        core_axis_name="c", subcore_axis_name="s",
        num_cores=NC, num_subcores=NS)

    @functools.partial(
        pl.kernel,
        out_type=jax.ShapeDtypeStruct((_WROWS, 128), jnp.float32),
        name="sc_table_relayout",
        mesh=mesh,
        scratch_types=[
            pltpu.VMEM((2, 2, 8, 128), jnp.float32),
            pltpu.VMEM((2, 2, 8, 128), jnp.float32),
            pltpu.SemaphoreType.DMA,
            pltpu.SemaphoreType.DMA,
        ],
        compiler_params=pltpu.CompilerParams(use_tc_tiling_on_sc=True,
                                             needs_layout_passes=False),
    )
    def k(t_hbm, tail_hbm, out_hbm, bin_v, bout_v, gsem, osem):
        wid = lax.axis_index("s") * NC + lax.axis_index("c")
        start = wid * _CPW_LO + jnp.minimum(wid, _CPW_REM)
        count = _CPW_LO + (wid < _CPW_REM).astype(jnp.int32)

        iot = lax.iota(jnp.int32, L)
        tv = iot // 8
        sv = iot % 8

        def fire_in(c, b):
            for h in range(2):
                pltpu.async_copy(
                    t_hbm.at[pl.ds(8 * h, 8), pl.ds(c * 128, 128)],
                    bin_v.at[b, h], gsem)

        def drain(sem, ref):
            pltpu.make_async_copy(t_hbm.at[pl.ds(0, 8), pl.ds(0, 128)],
                                  ref, sem).wait()

        def shuffle(b):
            for l in range(128):
                row = plsc.load_gather(
                    bin_v.at[b], [tv, sv, jnp.full((L,), l, jnp.int32)])
                bout_v[b, l // 64, (l // 8) % 8, pl.ds((l % 8) * 16, 16)] = row

        def fire_out(c, b):
            for h in range(2):
                pltpu.async_copy(
                    bout_v.at[b, h],
                    out_hbm.at[pl.ds(16 * c + 8 * h, 8), :], osem)

        fire_in(start, 0)

        def body(i, carry):
            b = i % 2
            fire_in(start + i + 1, 1 - b)
            for h in range(2):
                drain(gsem, bin_v.at[b, h])

            @pl.when(i >= 1)
            def _():
                for h in range(2):
                    drain(osem, bout_v.at[1 - b, h])

            shuffle(b)
            fire_out(start + i, b)
            return carry

        lax.fori_loop(0, count - 1, body, 0)

        last = count - 1
        bl = last % 2
        for h in range(2):
            drain(gsem, bin_v.at[bl, h])

        @pl.when(count >= 2)
        def _():
            for h in range(2):
                drain(osem, bout_v.at[1 - bl, h])

        shuffle(bl)
        fire_out(start + last, bl)
        for h in range(2):
            drain(osem, bout_v.at[bl, h])

        @pl.when(wid == NW - 1)
        def _():
            for h in range(2):
                pltpu.sync_copy(tail_hbm.at[pl.ds(8 * h, 8), :],
                                bin_v.at[0, h])
                pltpu.sync_copy(bin_v.at[0, h],
                                out_hbm.at[pl.ds(_TAILPAD // 8 + 8 * h, 8), :])

    return k(tableT, tail_wide)


def _sc_gather(x3d, table):
    mesh = plsc.VectorSubcoreMesh(
        core_axis_name="c", subcore_axis_name="s",
        num_cores=NC, num_subcores=NS)

    @functools.partial(
        pl.kernel,
        out_type=jax.ShapeDtypeStruct((ROWS, DIM), jnp.float32),
        name="sc_embed_gather",
        mesh=mesh,
        scratch_types=[
            pltpu.VMEM((SPW, STEP), jnp.int32),
            pltpu.VMEM((SUP, DIM), jnp.float32),
            pltpu.SemaphoreType.DMA,
        ],
        compiler_params=pltpu.CompilerParams(use_tc_tiling_on_sc=False),
    )
    def k(x_hbm, table_hbm, out_hbm, idx_v, rows_v, sem):
        wid = lax.axis_index("s") * NC + lax.axis_index("c")
        row_base = wid * RPW
        pltpu.sync_copy(x_hbm.at[wid], idx_v)

        lanes = lax.iota(jnp.int32, L)

        def to_idx(i, carry):
            for j in range(STEP // L):
                r0 = row_base + i * STEP + j * L
                prop = (r0 + lanes) % NPROP
                v = idx_v[i, pl.ds(j * L, L)] + prop * NVAL
                idx_v[i, pl.ds(j * L, L)] = jnp.where(
                    v >= _TAILBASE, v + (_TAILPAD - _TAILBASE), v)
            return carry

        lax.fori_loop(0, SPW, to_idx, 0)

        def sup(s, carry):
            cps = [
                pltpu.async_copy(
                    table_hbm.at[idx_v.at[s * KSTEP + j]],
                    rows_v.at[pl.ds(j * STEP, STEP)],
                    sem)
                for j in range(KSTEP)
            ]
            for c in cps:
                c.wait()
            pltpu.sync_copy(rows_v, out_hbm.at[pl.ds(row_base + s * SUP, SUP)])
            return carry

        lax.fori_loop(0, NSUP, sup, 0)

    return k(x3d, table)


_EG = (np.arange(NOBJ * NPROP)[:, None] // NPROP
       == np.arange(NOBJ)[None, :]).astype(np.float32)
_E16 = (np.arange(NOBJ)[:, None]
        == np.arange(FLAT)[None, :] // (NPROP * DIM)).astype(np.float32)
_E546 = _EG.T.copy()

_B_BLK = 128


def _tc_finish(raw2, x2, pattern, absent_t):
    grid = (BATCH // _B_BLK,)

    def body(raw_ref, x_ref, pat_ref, abs_ref, eg_ref, e16_ref, e546_ref,
             out_ref, pad_ref):
        xf = x_ref[...].astype(jnp.float32)
        sums = jnp.dot(xf, eg_ref[...], preferred_element_type=jnp.float32)
        padf = (sums == 0.0).astype(jnp.float32)
        m16 = jnp.dot(padf, e16_ref[...], preferred_element_type=jnp.float32)
        m546 = jnp.dot(padf, e546_ref[...], preferred_element_type=jnp.float32)
        emb = raw_ref[...] + pat_ref[...]
        out_ref[...] = emb * (1.0 - m16) + abs_ref[...] * m16
        pad_ref[...] = m546 > 0.5

    out2, padflat = pl.pallas_call(
        body,
        grid=grid,
        in_specs=[
            pl.BlockSpec((_B_BLK, FLAT), lambda i: (i, 0)),
            pl.BlockSpec((_B_BLK, NOBJ * NPROP), lambda i: (i, 0)),
            pl.BlockSpec((1, FLAT), lambda i: (0, 0)),
            pl.BlockSpec((1, FLAT), lambda i: (0, 0)),
            pl.BlockSpec((NOBJ * NPROP, NOBJ), lambda i: (0, 0)),
            pl.BlockSpec((NOBJ, FLAT), lambda i: (0, 0)),
            pl.BlockSpec((NOBJ, NOBJ * NPROP), lambda i: (0, 0)),
        ],
        out_specs=[
            pl.BlockSpec((_B_BLK, FLAT), lambda i: (i, 0)),
            pl.BlockSpec((_B_BLK, NOBJ * NPROP), lambda i: (i, 0)),
        ],
        out_shape=[
            jax.ShapeDtypeStruct((BATCH, FLAT), jnp.float32),
            jax.ShapeDtypeStruct((BATCH, NOBJ * NPROP), jnp.bool_),
        ],
    )(raw2, x2, pattern, absent_t, jnp.asarray(_EG), jnp.asarray(_E16),
      jnp.asarray(_E546))
    return out2, padflat


def kernel(table, mark_features, mark_objects, mark_absent, x):
    x3d = x.reshape(NW, SPW, STEP)
    tail_wide = jnp.pad(table[_TAILBASE:, :],
                        ((0, 128 - (_VOCAB - _TAILBASE)), (0, 0))
                        ).reshape(DIM, 128)
    table_wide = _sc_relayout(table.T, tail_wide)
    table_rm = table_wide.reshape(_VGATHER, DIM)
    raw = _sc_gather(x3d, table_rm)
    raw2 = raw.reshape(BATCH, FLAT)

    pattern = (mark_objects.reshape(NOBJ, 1, DIM)
               + mark_features.reshape(1, NPROP, DIM)).reshape(1, FLAT)
    absent_t = jnp.tile(mark_absent.reshape(1, DIM), (1, NOBJ * NPROP))
    x2 = x.reshape(BATCH, NOBJ * NPROP)

    out2, padflat = _tc_finish(raw2, x2, pattern, absent_t)
    return out2.reshape(BATCH, NOBJ * NPROP, DIM), padflat


# speedup vs baseline: 10.6479x; 1.6701x over previous
import functools

import numpy as np
import jax
import jax.numpy as jnp
from jax import lax
from jax.experimental import pallas as pl
from jax.experimental.pallas import tpu as pltpu
from jax.experimental.pallas import tpu_sc as plsc

DIM = 16
NPROP = 26
NOBJ = 21
NVAL = 100000
BATCH = 1024
ROWS = BATCH * NOBJ * NPROP
FLAT = NOBJ * NPROP * DIM

NC, NS, L = 2, 16, 16
NW = NC * NS
RPW = ROWS // NW
STEP = 96
SPW = RPW // STEP
KSTEP = 13
NSUP = SPW // KSTEP
SUP = KSTEP * STEP


_VOCAB = 1 + NVAL * NPROP
_NCOLS = _VOCAB // 128
_TAILBASE = _NCOLS * 128
_TAILPAD = _TAILBASE + 128
_WROWS = 325120
_VGATHER = _WROWS * 128 // DIM
_CPW_LO = _NCOLS // NW
_CPW_REM = _NCOLS - _CPW_LO * NW


def _sc_relayout(tableT, tail_wide):
    mesh = plsc.VectorSubcoreMesh(
        core_axis_name="c", subcore_axis_name="s",
        num_cores=NC, num_subcores=NS)

    @functools.partial(
        pl.kernel,
        out_type=jax.ShapeDtypeStruct((_WROWS, 128), jnp.float32),
        name="sc_table_relayout",
        mesh=mesh,
        scratch_types=[
            pltpu.VMEM((2, 2, 8, 128), jnp.float32),
            pltpu.VMEM((2, 2, 8, 128), jnp.float32),
            pltpu.SemaphoreType.DMA,
            pltpu.SemaphoreType.DMA,
        ],
        compiler_params=pltpu.CompilerParams(use_tc_tiling_on_sc=True,
                                             needs_layout_passes=False),
    )
    def k(t_hbm, tail_hbm, out_hbm, bin_v, bout_v, gsem, osem):
        wid = lax.axis_index("s") * NC + lax.axis_index("c")
        start = wid * _CPW_LO + jnp.minimum(wid, _CPW_REM)
        count = _CPW_LO + (wid < _CPW_REM).astype(jnp.int32)

        iot = lax.iota(jnp.int32, L)
        tv = iot // 8
        sv = iot % 8

        def fire_in(c, b):
            for h in range(2):
                pltpu.async_copy(
                    t_hbm.at[pl.ds(8 * h, 8), pl.ds(c * 128, 128)],
                    bin_v.at[b, h], gsem)

        def drain(sem, ref):
            pltpu.make_async_copy(t_hbm.at[pl.ds(0, 8), pl.ds(0, 128)],
                                  ref, sem).wait()

        def shuffle(b):
            for g in range(16):
                rows = [
                    plsc.load_gather(
                        bin_v.at[b],
                        [tv, sv, jnp.full((L,), 8 * g + j, jnp.int32)])
                    for j in range(8)
                ]
                for j in range(8):
                    l = 8 * g + j
                    bout_v[b, l // 64, (l // 8) % 8,
                           pl.ds((l % 8) * 16, 16)] = rows[j]

        def fire_out(c, b):
            for h in range(2):
                pltpu.async_copy(
                    bout_v.at[b, h],
                    out_hbm.at[pl.ds(16 * c + 8 * h, 8), :], osem)

        fire_in(start, 0)

        def body(i, carry):
            b = i % 2
            fire_in(start + i + 1, 1 - b)
            for h in range(2):
                drain(gsem, bin_v.at[b, h])

            @pl.when(i >= 1)
            def _():
                for h in range(2):
                    drain(osem, bout_v.at[1 - b, h])

            shuffle(b)
            fire_out(start + i, b)
            return carry

        lax.fori_loop(0, count - 1, body, 0)

        last = count - 1
        bl = last % 2
        for h in range(2):
            drain(gsem, bin_v.at[bl, h])

        @pl.when(count >= 2)
        def _():
            for h in range(2):
                drain(osem, bout_v.at[1 - bl, h])

        shuffle(bl)
        fire_out(start + last, bl)
        for h in range(2):
            drain(osem, bout_v.at[bl, h])

        @pl.when(wid == NW - 1)
        def _():
            for h in range(2):
                pltpu.sync_copy(tail_hbm.at[pl.ds(8 * h, 8), :],
                                bin_v.at[0, h])
                pltpu.sync_copy(bin_v.at[0, h],
                                out_hbm.at[pl.ds(_TAILPAD // 8 + 8 * h, 8), :])

    return k(tableT, tail_wide)


def _sc_gather(x3d, table):
    mesh = plsc.VectorSubcoreMesh(
        core_axis_name="c", subcore_axis_name="s",
        num_cores=NC, num_subcores=NS)

    @functools.partial(
        pl.kernel,
        out_type=jax.ShapeDtypeStruct((ROWS, DIM), jnp.float32),
        name="sc_embed_gather",
        mesh=mesh,
        scratch_types=[
            pltpu.VMEM((SPW, STEP), jnp.int32),
            pltpu.VMEM((SUP, DIM), jnp.float32),
            pltpu.SemaphoreType.DMA,
        ],
        compiler_params=pltpu.CompilerParams(use_tc_tiling_on_sc=False),
    )
    def k(x_hbm, table_hbm, out_hbm, idx_v, rows_v, sem):
        wid = lax.axis_index("s") * NC + lax.axis_index("c")
        row_base = wid * RPW
        pltpu.sync_copy(x_hbm.at[wid], idx_v)

        lanes = lax.iota(jnp.int32, L)

        def to_idx(i, carry):
            for j in range(STEP // L):
                r0 = row_base + i * STEP + j * L
                prop = (r0 + lanes) % NPROP
                v = idx_v[i, pl.ds(j * L, L)] + prop * NVAL
                idx_v[i, pl.ds(j * L, L)] = jnp.where(
                    v >= _TAILBASE, v + (_TAILPAD - _TAILBASE), v)
            return carry

        lax.fori_loop(0, SPW, to_idx, 0)

        def sup(s, carry):
            cps = [
                pltpu.async_copy(
                    table_hbm.at[idx_v.at[s * KSTEP + j]],
                    rows_v.at[pl.ds(j * STEP, STEP)],
                    sem)
                for j in range(KSTEP)
            ]
            for c in cps:
                c.wait()
            pltpu.sync_copy(rows_v, out_hbm.at[pl.ds(row_base + s * SUP, SUP)])
            return carry

        lax.fori_loop(0, NSUP, sup, 0)

    return k(x3d, table)


_EG = (np.arange(NOBJ * NPROP)[:, None] // NPROP
       == np.arange(NOBJ)[None, :]).astype(np.float32)
_E16 = (np.arange(NOBJ)[:, None]
        == np.arange(FLAT)[None, :] // (NPROP * DIM)).astype(np.float32)
_E546 = _EG.T.copy()

_B_BLK = 128


def _tc_finish(raw2, x2, pattern, absent_t):
    grid = (BATCH // _B_BLK,)

    def body(raw_ref, x_ref, pat_ref, abs_ref, eg_ref, e16_ref, e546_ref,
             out_ref, pad_ref):
        xf = x_ref[...].astype(jnp.float32)
        sums = jnp.dot(xf, eg_ref[...], preferred_element_type=jnp.float32)
        padf = (sums == 0.0).astype(jnp.float32)
        m16 = jnp.dot(padf, e16_ref[...], preferred_element_type=jnp.float32)
        m546 = jnp.dot(padf, e546_ref[...], preferred_element_type=jnp.float32)
        emb = raw_ref[...] + pat_ref[...]
        out_ref[...] = emb * (1.0 - m16) + abs_ref[...] * m16
        pad_ref[...] = m546 > 0.5

    out2, padflat = pl.pallas_call(
        body,
        grid=grid,
        in_specs=[
            pl.BlockSpec((_B_BLK, FLAT), lambda i: (i, 0)),
            pl.BlockSpec((_B_BLK, NOBJ * NPROP), lambda i: (i, 0)),
            pl.BlockSpec((1, FLAT), lambda i: (0, 0)),
            pl.BlockSpec((1, FLAT), lambda i: (0, 0)),
            pl.BlockSpec((NOBJ * NPROP, NOBJ), lambda i: (0, 0)),
            pl.BlockSpec((NOBJ, FLAT), lambda i: (0, 0)),
            pl.BlockSpec((NOBJ, NOBJ * NPROP), lambda i: (0, 0)),
        ],
        out_specs=[
            pl.BlockSpec((_B_BLK, FLAT), lambda i: (i, 0)),
            pl.BlockSpec((_B_BLK, NOBJ * NPROP), lambda i: (i, 0)),
        ],
        out_shape=[
            jax.ShapeDtypeStruct((BATCH, FLAT), jnp.float32),
            jax.ShapeDtypeStruct((BATCH, NOBJ * NPROP), jnp.bool_),
        ],
    )(raw2, x2, pattern, absent_t, jnp.asarray(_EG), jnp.asarray(_E16),
      jnp.asarray(_E546))
    return out2, padflat


def kernel(table, mark_features, mark_objects, mark_absent, x):
    x3d = x.reshape(NW, SPW, STEP)
    tail_wide = jnp.pad(table[_TAILBASE:, :],
                        ((0, 128 - (_VOCAB - _TAILBASE)), (0, 0))
                        ).reshape(DIM, 128)
    table_wide = _sc_relayout(table.T, tail_wide)
    table_rm = table_wide.reshape(_VGATHER, DIM)
    raw = _sc_gather(x3d, table_rm)
    raw2 = raw.reshape(BATCH, FLAT)

    pattern = (mark_objects.reshape(NOBJ, 1, DIM)
               + mark_features.reshape(1, NPROP, DIM)).reshape(1, FLAT)
    absent_t = jnp.tile(mark_absent.reshape(1, DIM), (1, NOBJ * NPROP))
    x2 = x.reshape(BATCH, NOBJ * NPROP)

    out2, padflat = _tc_finish(raw2, x2, pattern, absent_t)
    return out2.reshape(BATCH, NOBJ * NPROP, DIM), padflat


# speedup vs baseline: 11.9444x; 1.1218x over previous
import functools

import numpy as np
import jax
import jax.numpy as jnp
from jax import lax
from jax.experimental import pallas as pl
from jax.experimental.pallas import tpu as pltpu
from jax.experimental.pallas import tpu_sc as plsc

DIM = 16
NPROP = 26
NOBJ = 21
NVAL = 100000
BATCH = 1024
ROWS = BATCH * NOBJ * NPROP
FLAT = NOBJ * NPROP * DIM

NC, NS, L = 2, 16, 16
NW = NC * NS
RPW = ROWS // NW
STEP = 96
SPW = RPW // STEP
KSTEP = 13
NSUP = SPW // KSTEP
SUP = KSTEP * STEP


_VOCAB = 1 + NVAL * NPROP
_NCOLS = _VOCAB // 128
_TAILBASE = _NCOLS * 128
_TAILPAD = _TAILBASE + 128
_WROWS = 325120
_VGATHER = _WROWS * 128 // DIM
_CPW_LO = _NCOLS // NW
_CPW_REM = _NCOLS - _CPW_LO * NW


def _sc_relayout(tableT, tail_wide):
    mesh = plsc.VectorSubcoreMesh(
        core_axis_name="c", subcore_axis_name="s",
        num_cores=NC, num_subcores=NS)

    @functools.partial(
        pl.kernel,
        out_type=jax.ShapeDtypeStruct((_WROWS, 128), jnp.float32),
        name="sc_table_relayout",
        mesh=mesh,
        scratch_types=[
            pltpu.VMEM((4, 2, 8, 128), jnp.float32),
            pltpu.VMEM((4, 2, 8, 128), jnp.float32),
            pltpu.SemaphoreType.DMA,
            pltpu.SemaphoreType.DMA,
        ],
        compiler_params=pltpu.CompilerParams(use_tc_tiling_on_sc=True,
                                             needs_layout_passes=False),
    )
    def k(t_hbm, tail_hbm, out_hbm, bin_v, bout_v, gsem, osem):
        wid = lax.axis_index("s") * NC + lax.axis_index("c")
        start = wid * _CPW_LO + jnp.minimum(wid, _CPW_REM)
        count = _CPW_LO + (wid < _CPW_REM).astype(jnp.int32)

        iot = lax.iota(jnp.int32, L)
        tv = iot // 8
        sv = iot % 8

        def fire_in(c, b):
            for h in range(2):
                pltpu.async_copy(
                    t_hbm.at[pl.ds(8 * h, 8), pl.ds(c * 128, 128)],
                    bin_v.at[b, h], gsem)

        def drain(sem, ref):
            pltpu.make_async_copy(t_hbm.at[pl.ds(0, 8), pl.ds(0, 128)],
                                  ref, sem).wait()

        def shuffle(b):
            bb = bin_v.at[b]
            for g in range(8):
                rows = [
                    plsc.load_gather(
                        bb, [tv, sv, jnp.full((L,), 16 * g + j, jnp.int32)])
                    for j in range(16)
                ]
                for j in range(16):
                    l = 16 * g + j
                    bout_v[b, l // 64, (l // 8) % 8,
                           pl.ds((l % 8) * 16, 16)] = rows[j]

        def fire_out(c, b):
            for h in range(2):
                pltpu.async_copy(
                    bout_v.at[b, h],
                    out_hbm.at[pl.ds(16 * c + 8 * h, 8), :], osem)

        for q in range(3):
            fire_in(start + q, q)

        def body(i, carry):
            b = i % 4

            @pl.when(i + 3 < count)
            def _():
                fire_in(start + i + 3, (i + 3) % 4)

            for h in range(2):
                drain(gsem, bin_v.at[b, h])

            @pl.when(i >= 3)
            def _():
                for h in range(2):
                    drain(osem, bout_v.at[(i - 3) % 4, h])

            shuffle(b)
            fire_out(start + i, b)
            return carry

        lax.fori_loop(0, count, body, 0)

        for q in range(3):
            for h in range(2):
                drain(osem, bout_v.at[0, h])

        @pl.when(wid == NW - 1)
        def _():
            for h in range(2):
                pltpu.sync_copy(tail_hbm.at[pl.ds(8 * h, 8), :],
                                bin_v.at[0, h])
                pltpu.sync_copy(bin_v.at[0, h],
                                out_hbm.at[pl.ds(_TAILPAD // 8 + 8 * h, 8), :])

    return k(tableT, tail_wide)


def _sc_gather(x3d, table):
    mesh = plsc.VectorSubcoreMesh(
        core_axis_name="c", subcore_axis_name="s",
        num_cores=NC, num_subcores=NS)

    @functools.partial(
        pl.kernel,
        out_type=jax.ShapeDtypeStruct((ROWS, DIM), jnp.float32),
        name="sc_embed_gather",
        mesh=mesh,
        scratch_types=[
            pltpu.VMEM((SPW, STEP), jnp.int32),
            pltpu.VMEM((SUP, DIM), jnp.float32),
            pltpu.SemaphoreType.DMA,
        ],
        compiler_params=pltpu.CompilerParams(use_tc_tiling_on_sc=False),
    )
    def k(x_hbm, table_hbm, out_hbm, idx_v, rows_v, sem):
        wid = lax.axis_index("s") * NC + lax.axis_index("c")
        row_base = wid * RPW
        pltpu.sync_copy(x_hbm.at[wid], idx_v)

        lanes = lax.iota(jnp.int32, L)

        def to_idx(i, carry):
            for j in range(STEP // L):
                r0 = row_base + i * STEP + j * L
                prop = (r0 + lanes) % NPROP
                v = idx_v[i, pl.ds(j * L, L)] + prop * NVAL
                idx_v[i, pl.ds(j * L, L)] = jnp.where(
                    v >= _TAILBASE, v + (_TAILPAD - _TAILBASE), v)
            return carry

        lax.fori_loop(0, SPW, to_idx, 0)

        def sup(s, carry):
            cps = [
                pltpu.async_copy(
                    table_hbm.at[idx_v.at[s * KSTEP + j]],
                    rows_v.at[pl.ds(j * STEP, STEP)],
                    sem)
                for j in range(KSTEP)
            ]
            for c in cps:
                c.wait()
            pltpu.sync_copy(rows_v, out_hbm.at[pl.ds(row_base + s * SUP, SUP)])
            return carry

        lax.fori_loop(0, NSUP, sup, 0)

    return k(x3d, table)


_EG = (np.arange(NOBJ * NPROP)[:, None] // NPROP
       == np.arange(NOBJ)[None, :]).astype(np.float32)
_E16 = (np.arange(NOBJ)[:, None]
        == np.arange(FLAT)[None, :] // (NPROP * DIM)).astype(np.float32)
_E546 = _EG.T.copy()

_B_BLK = 128


def _tc_finish(raw2, x2, pattern, absent_t):
    grid = (BATCH // _B_BLK,)

    def body(raw_ref, x_ref, pat_ref, abs_ref, eg_ref, e16_ref, e546_ref,
             out_ref, pad_ref):
        xf = x_ref[...].astype(jnp.float32)
        sums = jnp.dot(xf, eg_ref[...], preferred_element_type=jnp.float32)
        padf = (sums == 0.0).astype(jnp.float32)
        m16 = jnp.dot(padf, e16_ref[...], preferred_element_type=jnp.float32)
        m546 = jnp.dot(padf, e546_ref[...], preferred_element_type=jnp.float32)
        emb = raw_ref[...] + pat_ref[...]
        out_ref[...] = emb * (1.0 - m16) + abs_ref[...] * m16
        pad_ref[...] = m546 > 0.5

    out2, padflat = pl.pallas_call(
        body,
        grid=grid,
        in_specs=[
            pl.BlockSpec((_B_BLK, FLAT), lambda i: (i, 0)),
            pl.BlockSpec((_B_BLK, NOBJ * NPROP), lambda i: (i, 0)),
            pl.BlockSpec((1, FLAT), lambda i: (0, 0)),
            pl.BlockSpec((1, FLAT), lambda i: (0, 0)),
            pl.BlockSpec((NOBJ * NPROP, NOBJ), lambda i: (0, 0)),
            pl.BlockSpec((NOBJ, FLAT), lambda i: (0, 0)),
            pl.BlockSpec((NOBJ, NOBJ * NPROP), lambda i: (0, 0)),
        ],
        out_specs=[
            pl.BlockSpec((_B_BLK, FLAT), lambda i: (i, 0)),
            pl.BlockSpec((_B_BLK, NOBJ * NPROP), lambda i: (i, 0)),
        ],
        out_shape=[
            jax.ShapeDtypeStruct((BATCH, FLAT), jnp.float32),
            jax.ShapeDtypeStruct((BATCH, NOBJ * NPROP), jnp.bool_),
        ],
    )(raw2, x2, pattern, absent_t, jnp.asarray(_EG), jnp.asarray(_E16),
      jnp.asarray(_E546))
    return out2, padflat


def kernel(table, mark_features, mark_objects, mark_absent, x):
    x3d = x.reshape(NW, SPW, STEP)
    tail_wide = jnp.pad(table[_TAILBASE:, :],
                        ((0, 128 - (_VOCAB - _TAILBASE)), (0, 0))
                        ).reshape(DIM, 128)
    table_wide = _sc_relayout(table.T, tail_wide)
    table_rm = table_wide.reshape(_VGATHER, DIM)
    raw = _sc_gather(x3d, table_rm)
    raw2 = raw.reshape(BATCH, FLAT)

    pattern = (mark_objects.reshape(NOBJ, 1, DIM)
               + mark_features.reshape(1, NPROP, DIM)).reshape(1, FLAT)
    absent_t = jnp.tile(mark_absent.reshape(1, DIM), (1, NOBJ * NPROP))
    x2 = x.reshape(BATCH, NOBJ * NPROP)

    out2, padflat = _tc_finish(raw2, x2, pattern, absent_t)
    return out2.reshape(BATCH, NOBJ * NPROP, DIM), padflat


# speedup vs baseline: 11.9779x; 1.0028x over previous
import functools

import numpy as np
import jax
import jax.numpy as jnp
from jax import lax
from jax.experimental import pallas as pl
from jax.experimental.pallas import tpu as pltpu
from jax.experimental.pallas import tpu_sc as plsc

DIM = 16
NPROP = 26
NOBJ = 21
NVAL = 100000
BATCH = 1024
ROWS = BATCH * NOBJ * NPROP
FLAT = NOBJ * NPROP * DIM

NC, NS, L = 2, 16, 16
NW = NC * NS
RPW = ROWS // NW
STEP = 96
SPW = RPW // STEP
KSTEP = 13
NSUP = SPW // KSTEP
SUP = KSTEP * STEP


_VOCAB = 1 + NVAL * NPROP
_NCOLS = _VOCAB // 128
_TAILBASE = _NCOLS * 128
_TAILPAD = _TAILBASE + 128
_WROWS = 325120
_VGATHER = _WROWS * 128 // DIM
_CPW_LO = _NCOLS // NW
_CPW_REM = _NCOLS - _CPW_LO * NW


def _sc_relayout(tableT, tail_wide):
    mesh = plsc.VectorSubcoreMesh(
        core_axis_name="c", subcore_axis_name="s",
        num_cores=NC, num_subcores=NS)

    NB = 8

    @functools.partial(
        pl.kernel,
        out_type=jax.ShapeDtypeStruct((_WROWS // 8, 8, 128), jnp.float32),
        name="sc_table_relayout",
        mesh=mesh,
        scratch_types=[
            pltpu.VMEM((8, 2, 8, 128), jnp.float32),
            pltpu.VMEM((8, 2, 8, 128), jnp.float32),
            pltpu.SemaphoreType.DMA,
            pltpu.SemaphoreType.DMA,
        ],
        compiler_params=pltpu.CompilerParams(use_tc_tiling_on_sc=True,
                                             needs_layout_passes=False),
    )
    def k(t_hbm, tail_hbm, out_hbm, bin_v, bout_v, gsem, osem):
        wid = lax.axis_index("s") * NC + lax.axis_index("c")
        start = wid * _CPW_LO + jnp.minimum(wid, _CPW_REM)
        count = _CPW_LO + (wid < _CPW_REM).astype(jnp.int32)

        iot = lax.iota(jnp.int32, L)
        tv = iot // 8
        sv = iot % 8

        def fire_in(c, b):
            for h in range(2):
                pltpu.async_copy(
                    t_hbm.at[pl.ds(8 * h, 8), pl.ds(c * 128, 128)],
                    bin_v.at[b, h], gsem)

        def drain_in(b):
            for h in range(2):
                pltpu.make_async_copy(
                    t_hbm.at[pl.ds(0, 8), pl.ds(0, 128)],
                    bin_v.at[b, h], gsem).wait()

        def drain_out(b):
            pltpu.make_async_copy(out_hbm.at[pl.ds(0, 2)],
                                  bout_v.at[b], osem).wait()

        def shuffle(b):
            bb = bin_v.at[b]
            for g in range(8):
                rows = [
                    plsc.load_gather(
                        bb, [tv, sv, jnp.full((L,), 16 * g + j, jnp.int32)])
                    for j in range(16)
                ]
                for j in range(16):
                    l = 16 * g + j
                    bout_v[b, l // 64, (l // 8) % 8,
                           pl.ds((l % 8) * 16, 16)] = rows[j]

        def fire_out(c, b):
            pltpu.async_copy(bout_v.at[b], out_hbm.at[pl.ds(2 * c, 2)], osem)

        for q in range(NB - 1):
            fire_in(start + q, q)

        def body(i, carry):
            b = i % NB

            @pl.when(i + (NB - 1) < count)
            def _():
                fire_in(start + i + (NB - 1), (i + (NB - 1)) % NB)

            drain_in(b)

            @pl.when(i >= NB - 1)
            def _():
                drain_out((i - (NB - 1)) % NB)

            shuffle(b)
            fire_out(start + i, b)
            return carry

        lax.fori_loop(0, count, body, 0)

        for q in range(NB - 1):
            drain_out(0)

        @pl.when(wid == NW - 1)
        def _():
            for h in range(2):
                pltpu.sync_copy(tail_hbm.at[pl.ds(8 * h, 8), :],
                                bin_v.at[0, h])
            pltpu.sync_copy(bin_v.at[0], out_hbm.at[pl.ds(_TAILPAD // 64, 2)])

    return k(tableT, tail_wide)


def _sc_gather(x3d, table):
    mesh = plsc.VectorSubcoreMesh(
        core_axis_name="c", subcore_axis_name="s",
        num_cores=NC, num_subcores=NS)

    @functools.partial(
        pl.kernel,
        out_type=jax.ShapeDtypeStruct((ROWS, DIM), jnp.float32),
        name="sc_embed_gather",
        mesh=mesh,
        scratch_types=[
            pltpu.VMEM((SPW, STEP), jnp.int32),
            pltpu.VMEM((SUP, DIM), jnp.float32),
            pltpu.SemaphoreType.DMA,
        ],
        compiler_params=pltpu.CompilerParams(use_tc_tiling_on_sc=False),
    )
    def k(x_hbm, table_hbm, out_hbm, idx_v, rows_v, sem):
        wid = lax.axis_index("s") * NC + lax.axis_index("c")
        row_base = wid * RPW
        pltpu.sync_copy(x_hbm.at[wid], idx_v)

        lanes = lax.iota(jnp.int32, L)

        def to_idx(i, carry):
            for j in range(STEP // L):
                r0 = row_base + i * STEP + j * L
                prop = (r0 + lanes) % NPROP
                v = idx_v[i, pl.ds(j * L, L)] + prop * NVAL
                idx_v[i, pl.ds(j * L, L)] = jnp.where(
                    v >= _TAILBASE, v + (_TAILPAD - _TAILBASE), v)
            return carry

        lax.fori_loop(0, SPW, to_idx, 0)

        def sup(s, carry):
            cps = [
                pltpu.async_copy(
                    table_hbm.at[idx_v.at[s * KSTEP + j]],
                    rows_v.at[pl.ds(j * STEP, STEP)],
                    sem)
                for j in range(KSTEP)
            ]
            for c in cps:
                c.wait()
            pltpu.sync_copy(rows_v, out_hbm.at[pl.ds(row_base + s * SUP, SUP)])
            return carry

        lax.fori_loop(0, NSUP, sup, 0)

    return k(x3d, table)


_EG = (np.arange(NOBJ * NPROP)[:, None] // NPROP
       == np.arange(NOBJ)[None, :]).astype(np.float32)
_E16 = (np.arange(NOBJ)[:, None]
        == np.arange(FLAT)[None, :] // (NPROP * DIM)).astype(np.float32)
_E546 = _EG.T.copy()

_B_BLK = 128


def _tc_finish(raw2, x2, pattern, absent_t):
    grid = (BATCH // _B_BLK,)

    def body(raw_ref, x_ref, pat_ref, abs_ref, eg_ref, e16_ref, e546_ref,
             out_ref, pad_ref):
        xf = x_ref[...].astype(jnp.float32)
        sums = jnp.dot(xf, eg_ref[...], preferred_element_type=jnp.float32)
        padf = (sums == 0.0).astype(jnp.float32)
        m16 = jnp.dot(padf, e16_ref[...], preferred_element_type=jnp.float32)
        m546 = jnp.dot(padf, e546_ref[...], preferred_element_type=jnp.float32)
        emb = raw_ref[...] + pat_ref[...]
        out_ref[...] = emb * (1.0 - m16) + abs_ref[...] * m16
        pad_ref[...] = m546 > 0.5

    out2, padflat = pl.pallas_call(
        body,
        grid=grid,
        in_specs=[
            pl.BlockSpec((_B_BLK, FLAT), lambda i: (i, 0)),
            pl.BlockSpec((_B_BLK, NOBJ * NPROP), lambda i: (i, 0)),
            pl.BlockSpec((1, FLAT), lambda i: (0, 0)),
            pl.BlockSpec((1, FLAT), lambda i: (0, 0)),
            pl.BlockSpec((NOBJ * NPROP, NOBJ), lambda i: (0, 0)),
            pl.BlockSpec((NOBJ, FLAT), lambda i: (0, 0)),
            pl.BlockSpec((NOBJ, NOBJ * NPROP), lambda i: (0, 0)),
        ],
        out_specs=[
            pl.BlockSpec((_B_BLK, FLAT), lambda i: (i, 0)),
            pl.BlockSpec((_B_BLK, NOBJ * NPROP), lambda i: (i, 0)),
        ],
        out_shape=[
            jax.ShapeDtypeStruct((BATCH, FLAT), jnp.float32),
            jax.ShapeDtypeStruct((BATCH, NOBJ * NPROP), jnp.bool_),
        ],
    )(raw2, x2, pattern, absent_t, jnp.asarray(_EG), jnp.asarray(_E16),
      jnp.asarray(_E546))
    return out2, padflat


def kernel(table, mark_features, mark_objects, mark_absent, x):
    x3d = x.reshape(NW, SPW, STEP)
    tail_wide = jnp.pad(table[_TAILBASE:, :],
                        ((0, 128 - (_VOCAB - _TAILBASE)), (0, 0))
                        ).reshape(DIM, 128)
    table_wide = _sc_relayout(table.T, tail_wide)
    table_rm = table_wide.reshape(_VGATHER, DIM)
    raw = _sc_gather(x3d, table_rm)
    raw2 = raw.reshape(BATCH, FLAT)

    pattern = (mark_objects.reshape(NOBJ, 1, DIM)
               + mark_features.reshape(1, NPROP, DIM)).reshape(1, FLAT)
    absent_t = jnp.tile(mark_absent.reshape(1, DIM), (1, NOBJ * NPROP))
    x2 = x.reshape(BATCH, NOBJ * NPROP)

    out2, padflat = _tc_finish(raw2, x2, pattern, absent_t)
    return out2.reshape(BATCH, NOBJ * NPROP, DIM), padflat


# speedup vs baseline: 23.1543x; 1.9331x over previous
import functools

import numpy as np
import jax
import jax.numpy as jnp
from jax import lax
from jax.experimental import pallas as pl
from jax.experimental.pallas import tpu as pltpu
from jax.experimental.pallas import tpu_sc as plsc

DIM = 16
NPROP = 26
NOBJ = 21
NVAL = 100000
BATCH = 1024
ROWS = BATCH * NOBJ * NPROP
FLAT = NOBJ * NPROP * DIM

NC, NS, L = 2, 16, 16
NW = NC * NS
RPW = ROWS // NW
STEP = 96
SPW = RPW // STEP
KSTEP = 13
NSUP = SPW // KSTEP
SUP = KSTEP * STEP


_VOCAB = 1 + NVAL * NPROP
_NCOLS = _VOCAB // 128
_TAILBASE = _NCOLS * 128
_TAILPAD = _TAILBASE + 128
_WROWS = 325120
_VGATHER = _WROWS * 128 // DIM
_CPW_LO = _NCOLS // NW
_CPW_REM = _NCOLS - _CPW_LO * NW


def _sc_relayout(tableT, tail_wide):
    mesh = plsc.VectorSubcoreMesh(
        core_axis_name="c", subcore_axis_name="s",
        num_cores=NC, num_subcores=NS)

    NB = 8

    @functools.partial(
        pl.kernel,
        out_type=jax.ShapeDtypeStruct((_WROWS // 8, 8, 128), jnp.float32),
        name="sc_table_relayout",
        mesh=mesh,
        scratch_types=[
            pltpu.VMEM((8, 2, 8, 128), jnp.float32),
            pltpu.VMEM((8, 2, 8, 128), jnp.float32),
            pltpu.SemaphoreType.DMA,
            pltpu.SemaphoreType.DMA,
        ],
        compiler_params=pltpu.CompilerParams(use_tc_tiling_on_sc=True,
                                             needs_layout_passes=False),
    )
    def k(t_hbm, tail_hbm, out_hbm, bin_v, bout_v, gsem, osem):
        wid = lax.axis_index("s") * NC + lax.axis_index("c")
        start = wid * _CPW_LO + jnp.minimum(wid, _CPW_REM)
        count = _CPW_LO + (wid < _CPW_REM).astype(jnp.int32)

        iot = lax.iota(jnp.int32, L)
        sv = iot % 8
        cv = iot // 8

        def fire_in(c, b):
            for h in range(2):
                pltpu.async_copy(
                    t_hbm.at[pl.ds(8 * h, 8), pl.ds(c * 128, 128)],
                    bin_v.at[b, h], gsem)

        def drain_in(b):
            for h in range(2):
                pltpu.make_async_copy(
                    t_hbm.at[pl.ds(0, 8), pl.ds(0, 128)],
                    bin_v.at[b, h], gsem).wait()

        def drain_out(b):
            pltpu.make_async_copy(out_hbm.at[pl.ds(0, 2)],
                                  bout_v.at[b], osem).wait()

        def shuffle(b):
            for h in range(2):
                bb = bin_v.at[b, h]
                for g in range(4):
                    rows = [
                        plsc.load_gather(bb, [sv, cv + (32 * g + 2 * j)])
                        for j in range(16)
                    ]
                    for j in range(16):
                        l = 32 * g + 2 * j
                        bout_v[b, l // 64, (l // 8) % 8,
                               pl.ds((l % 8) * 16 + 8 * h, 16)] = rows[j]

        def fire_out(c, b):
            pltpu.async_copy(bout_v.at[b], out_hbm.at[pl.ds(2 * c, 2)], osem)

        for q in range(NB - 1):
            fire_in(start + q, q)

        def body(i, carry):
            b = i % NB

            @pl.when(i + (NB - 1) < count)
            def _():
                fire_in(start + i + (NB - 1), (i + (NB - 1)) % NB)

            drain_in(b)

            @pl.when(i >= NB - 1)
            def _():
                drain_out((i - (NB - 1)) % NB)

            shuffle(b)
            fire_out(start + i, b)
            return carry

        lax.fori_loop(0, count, body, 0)

        for q in range(NB - 1):
            drain_out(0)

        @pl.when(wid == NW - 1)
        def _():
            for h in range(2):
                pltpu.sync_copy(tail_hbm.at[pl.ds(8 * h, 8), :],
                                bin_v.at[0, h])
            pltpu.sync_copy(bin_v.at[0], out_hbm.at[pl.ds(_TAILPAD // 64, 2)])

    return k(tableT, tail_wide)


def _sc_gather(x3d, table):
    mesh = plsc.VectorSubcoreMesh(
        core_axis_name="c", subcore_axis_name="s",
        num_cores=NC, num_subcores=NS)

    @functools.partial(
        pl.kernel,
        out_type=jax.ShapeDtypeStruct((ROWS, DIM), jnp.float32),
        name="sc_embed_gather",
        mesh=mesh,
        scratch_types=[
            pltpu.VMEM((SPW, STEP), jnp.int32),
            pltpu.VMEM((SUP, DIM), jnp.float32),
            pltpu.SemaphoreType.DMA,
        ],
        compiler_params=pltpu.CompilerParams(use_tc_tiling_on_sc=False),
    )
    def k(x_hbm, table_hbm, out_hbm, idx_v, rows_v, sem):
        wid = lax.axis_index("s") * NC + lax.axis_index("c")
        row_base = wid * RPW
        pltpu.sync_copy(x_hbm.at[wid], idx_v)

        lanes = lax.iota(jnp.int32, L)

        def to_idx(i, carry):
            for j in range(STEP // L):
                r0 = row_base + i * STEP + j * L
                prop = (r0 + lanes) % NPROP
                v = idx_v[i, pl.ds(j * L, L)] + prop * NVAL
                idx_v[i, pl.ds(j * L, L)] = jnp.where(
                    v >= _TAILBASE, v + (_TAILPAD - _TAILBASE), v)
            return carry

        lax.fori_loop(0, SPW, to_idx, 0)

        def sup(s, carry):
            cps = [
                pltpu.async_copy(
                    table_hbm.at[idx_v.at[s * KSTEP + j]],
                    rows_v.at[pl.ds(j * STEP, STEP)],
                    sem)
                for j in range(KSTEP)
            ]
            for c in cps:
                c.wait()
            pltpu.sync_copy(rows_v, out_hbm.at[pl.ds(row_base + s * SUP, SUP)])
            return carry

        lax.fori_loop(0, NSUP, sup, 0)

    return k(x3d, table)


_EG = (np.arange(NOBJ * NPROP)[:, None] // NPROP
       == np.arange(NOBJ)[None, :]).astype(np.float32)
_E16 = (np.arange(NOBJ)[:, None]
        == np.arange(FLAT)[None, :] // (NPROP * DIM)).astype(np.float32)
_E546 = _EG.T.copy()

_B_BLK = 128


def _tc_finish(raw2, x2, pattern, absent_t):
    grid = (BATCH // _B_BLK,)

    def body(raw_ref, x_ref, pat_ref, abs_ref, eg_ref, e16_ref, e546_ref,
             out_ref, pad_ref):
        xf = x_ref[...].astype(jnp.float32)
        sums = jnp.dot(xf, eg_ref[...], preferred_element_type=jnp.float32)
        padf = (sums == 0.0).astype(jnp.float32)
        m16 = jnp.dot(padf, e16_ref[...], preferred_element_type=jnp.float32)
        m546 = jnp.dot(padf, e546_ref[...], preferred_element_type=jnp.float32)
        emb = raw_ref[...] + pat_ref[...]
        out_ref[...] = emb * (1.0 - m16) + abs_ref[...] * m16
        pad_ref[...] = m546 > 0.5

    out2, padflat = pl.pallas_call(
        body,
        grid=grid,
        in_specs=[
            pl.BlockSpec((_B_BLK, FLAT), lambda i: (i, 0)),
            pl.BlockSpec((_B_BLK, NOBJ * NPROP), lambda i: (i, 0)),
            pl.BlockSpec((1, FLAT), lambda i: (0, 0)),
            pl.BlockSpec((1, FLAT), lambda i: (0, 0)),
            pl.BlockSpec((NOBJ * NPROP, NOBJ), lambda i: (0, 0)),
            pl.BlockSpec((NOBJ, FLAT), lambda i: (0, 0)),
            pl.BlockSpec((NOBJ, NOBJ * NPROP), lambda i: (0, 0)),
        ],
        out_specs=[
            pl.BlockSpec((_B_BLK, FLAT), lambda i: (i, 0)),
            pl.BlockSpec((_B_BLK, NOBJ * NPROP), lambda i: (i, 0)),
        ],
        out_shape=[
            jax.ShapeDtypeStruct((BATCH, FLAT), jnp.float32),
            jax.ShapeDtypeStruct((BATCH, NOBJ * NPROP), jnp.bool_),
        ],
    )(raw2, x2, pattern, absent_t, jnp.asarray(_EG), jnp.asarray(_E16),
      jnp.asarray(_E546))
    return out2, padflat


def kernel(table, mark_features, mark_objects, mark_absent, x):
    x3d = x.reshape(NW, SPW, STEP)
    tail_wide = jnp.pad(table[_TAILBASE:, :],
                        ((0, 128 - (_VOCAB - _TAILBASE)), (0, 0))
                        ).reshape(DIM, 128)
    table_wide = _sc_relayout(table.T, tail_wide)
    table_rm = table_wide.reshape(_VGATHER, DIM)
    raw = _sc_gather(x3d, table_rm)
    raw2 = raw.reshape(BATCH, FLAT)

    pattern = (mark_objects.reshape(NOBJ, 1, DIM)
               + mark_features.reshape(1, NPROP, DIM)).reshape(1, FLAT)
    absent_t = jnp.tile(mark_absent.reshape(1, DIM), (1, NOBJ * NPROP))
    x2 = x.reshape(BATCH, NOBJ * NPROP)

    out2, padflat = _tc_finish(raw2, x2, pattern, absent_t)
    return out2.reshape(BATCH, NOBJ * NPROP, DIM), padflat
